# baseline (speedup 1.0000x reference)
import functools

import jax
import jax.numpy as jnp
from jax.experimental import pallas as pl
from jax.experimental.pallas import tpu as pltpu

D_IN = 2048
D_OUT = 2048
R = 64
E = 8
ALPHA = 16.0
SCALING = ALPHA / R
TILE = 1024

_CONTRACT_LAST = (((1,), (1,)), ((), ()))


def _adamole_kernel(x_ref, rw_ref, rb_ref, a_ref, b_ref, out_ref, bt_ref):
    @pl.when(pl.program_id(0) == 0)
    def _():
        for e in range(E):
            bt_ref[e * R:(e + 1) * R, :] = b_ref[e].T

    x = x_ref[...]

    rt = jax.lax.dot_general(x, rw_ref[...], _CONTRACT_LAST,
                             preferred_element_type=jnp.float32)
    rt = rt + rb_ref[...]
    logits = rt[:, :E]
    thr_lin = rt[:, E:E + 1]
    z = jax.lax.dot_general(x, a_ref[...], _CONTRACT_LAST,
                            preferred_element_type=jnp.float32)

    m = jnp.max(logits, axis=-1, keepdims=True)
    ex = jnp.exp(logits - m)
    gate = ex / jnp.sum(ex, axis=-1, keepdims=True)
    thr = jax.nn.sigmoid(thr_lin) * (1.0 / E)
    adapted = gate - thr
    w = jnp.where(adapted >= 0.0, adapted, 0.0)
    s = jnp.sum(w, axis=-1, keepdims=True)
    s = jnp.where(s == 0.0, 1.0, s)
    w = w / s

    erow = jax.lax.broadcasted_iota(jnp.int32, (E, E * R), 0)
    ecol = jax.lax.broadcasted_iota(jnp.int32, (E, E * R), 1) // R
    expand = jnp.where(erow == ecol, 1.0, 0.0).astype(jnp.float32)
    w_exp = jnp.dot(w, expand, preferred_element_type=jnp.float32)

    zw = z * w_exp
    out = jnp.dot(zw, bt_ref[...], preferred_element_type=jnp.float32)
    out_ref[...] = out * SCALING


@functools.partial(jax.jit, static_argnames=())
def _adamole(flat, rw_cat, rb_cat, lora_A2, b_flat):
    t = flat.shape[0]
    grid = (t // TILE,)
    return pl.pallas_call(
        _adamole_kernel,
        grid=grid,
        in_specs=[
            pl.BlockSpec((TILE, D_IN), lambda i: (i, 0)),
            pl.BlockSpec((E + 1, D_IN), lambda i: (0, 0)),
            pl.BlockSpec((1, E + 1), lambda i: (0, 0)),
            pl.BlockSpec((E * R, D_IN), lambda i: (0, 0)),
            pl.BlockSpec((E, D_OUT, R), lambda i: (0, 0, 0)),
        ],
        out_specs=pl.BlockSpec((TILE, D_OUT), lambda i: (i, 0)),
        out_shape=jax.ShapeDtypeStruct((t, D_OUT), jnp.float32),
        scratch_shapes=[pltpu.VMEM((E * R, D_OUT), jnp.float32)],
        compiler_params=pltpu.CompilerParams(
            dimension_semantics=("arbitrary",),
        ),
    )(flat, rw_cat, rb_cat, lora_A2, b_flat)


def kernel(inputs, router_W, router_b, thr_W, thr_b, lora_A, lora_B):
    flat = inputs.reshape(-1, inputs.shape[-1])
    rw_cat = jnp.concatenate([router_W, thr_W], axis=0)
    rb_cat = jnp.concatenate([router_b, thr_b], axis=0).reshape(1, E + 1)
    out = _adamole(flat, rw_cat, rb_cat, lora_A.reshape(E * R, D_IN), lora_B)
    return out.reshape((*inputs.shape[:-1], D_OUT))


# speedup vs baseline: 1.0936x; 1.0936x over previous
import functools

import jax
import jax.numpy as jnp
from jax.experimental import pallas as pl
from jax.experimental.pallas import tpu as pltpu

D_IN = 2048
D_OUT = 2048
R = 64
E = 8
ALPHA = 16.0
SCALING = ALPHA / R
TILE = 1024

_CONTRACT_LAST = (((1,), (1,)), ((), ()))


def _adamole_kernel(x_ref, rw_ref, rb_ref, a_ref, b_ref, out_ref):
    x = x_ref[...]

    rt = jax.lax.dot_general(x, rw_ref[...], _CONTRACT_LAST,
                             preferred_element_type=jnp.float32)
    rt = rt + rb_ref[...]
    logits = rt[:, :E]
    thr_lin = rt[:, E:E + 1]
    z = jax.lax.dot_general(x, a_ref[...], _CONTRACT_LAST,
                            preferred_element_type=jnp.float32)

    m = jnp.max(logits, axis=-1, keepdims=True)
    ex = jnp.exp(logits - m)
    gate = ex / jnp.sum(ex, axis=-1, keepdims=True)
    thr = jax.nn.sigmoid(thr_lin) * (1.0 / E)
    adapted = gate - thr
    w = jnp.where(adapted >= 0.0, adapted, 0.0)
    s = jnp.sum(w, axis=-1, keepdims=True)
    s = jnp.where(s == 0.0, 1.0, s)
    w = w / s

    erow = jax.lax.broadcasted_iota(jnp.int32, (E, E * R), 0)
    ecol = jax.lax.broadcasted_iota(jnp.int32, (E, E * R), 1) // R
    expand = jnp.where(erow == ecol, 1.0, 0.0).astype(jnp.float32)
    w_exp = jnp.dot(w, expand, preferred_element_type=jnp.float32)

    zw = (z * w_exp).astype(jnp.bfloat16)
    out = jnp.dot(zw, b_ref[...], preferred_element_type=jnp.float32)
    out_ref[...] = out * SCALING


@functools.partial(jax.jit, static_argnames=())
def _adamole(flat, rw_cat, rb_cat, lora_A2, b_flat):
    t = flat.shape[0]
    grid = (t // TILE,)
    return pl.pallas_call(
        _adamole_kernel,
        grid=grid,
        in_specs=[
            pl.BlockSpec((TILE, D_IN), lambda i: (i, 0)),
            pl.BlockSpec((E + 1, D_IN), lambda i: (0, 0)),
            pl.BlockSpec((1, E + 1), lambda i: (0, 0)),
            pl.BlockSpec((E * R, D_IN), lambda i: (0, 0)),
            pl.BlockSpec((E * R, D_OUT), lambda i: (0, 0)),
        ],
        out_specs=pl.BlockSpec((TILE, D_OUT), lambda i: (i, 0)),
        out_shape=jax.ShapeDtypeStruct((t, D_OUT), jnp.float32),
        compiler_params=pltpu.CompilerParams(
            dimension_semantics=("parallel",),
        ),
    )(flat, rw_cat, rb_cat, lora_A2, b_flat)


def kernel(inputs, router_W, router_b, thr_W, thr_b, lora_A, lora_B):
    flat = inputs.reshape(-1, inputs.shape[-1])
    rw_cat = jnp.concatenate([router_W, thr_W], axis=0)
    rb_cat = jnp.concatenate([router_b, thr_b], axis=0).reshape(1, E + 1)
    b_flat = jnp.transpose(lora_B.astype(jnp.bfloat16), (0, 2, 1)
                           ).reshape(E * R, D_OUT)
    out = _adamole(flat, rw_cat, rb_cat, lora_A.reshape(E * R, D_IN), b_flat)
    return out.reshape((*inputs.shape[:-1], D_OUT))


# speedup vs baseline: 1.1472x; 1.0490x over previous
import functools

import jax
import jax.numpy as jnp
from jax.experimental import pallas as pl
from jax.experimental.pallas import tpu as pltpu

D_IN = 2048
D_OUT = 2048
R = 64
E = 8
ALPHA = 16.0
SCALING = ALPHA / R
TILE = 1024

_CONTRACT_LAST = (((1,), (1,)), ((), ()))


def _adamole_kernel(x_ref, rw_ref, rb_ref, a_ref, b_ref, out_ref):
    x = x_ref[...]

    rt = jax.lax.dot_general(x, rw_ref[...], _CONTRACT_LAST,
                             preferred_element_type=jnp.float32)
    rt = rt + rb_ref[...]
    logits = rt[:, :E]
    thr_lin = rt[:, E:E + 1]
    z = jax.lax.dot_general(x, a_ref[...], _CONTRACT_LAST,
                            preferred_element_type=jnp.float32)

    m = jnp.max(logits, axis=-1, keepdims=True)
    ex = jnp.exp(logits - m)
    gate = ex / jnp.sum(ex, axis=-1, keepdims=True)
    thr = jax.nn.sigmoid(thr_lin) * (1.0 / E)
    adapted = gate - thr
    w = jnp.where(adapted >= 0.0, adapted, 0.0)
    s = jnp.sum(w, axis=-1, keepdims=True)
    s = jnp.where(s == 0.0, 1.0, s)
    w = w / s

    erow = jax.lax.broadcasted_iota(jnp.int32, (E, E * R), 0)
    ecol = jax.lax.broadcasted_iota(jnp.int32, (E, E * R), 1) // R
    expand = jnp.where(erow == ecol, 1.0, 0.0).astype(jnp.float32)
    w_exp = jnp.dot(w, expand, preferred_element_type=jnp.float32)

    zw = z * w_exp
    out = jnp.dot(zw, b_ref[...], preferred_element_type=jnp.float32)
    out_ref[...] = out * SCALING


@functools.partial(jax.jit, static_argnames=())
def _adamole(flat, rw_cat, rb_cat, lora_A2, b_flat):
    t = flat.shape[0]
    grid = (t // TILE,)
    return pl.pallas_call(
        _adamole_kernel,
        grid=grid,
        in_specs=[
            pl.BlockSpec((TILE, D_IN), lambda i: (i, 0)),
            pl.BlockSpec((E + 1, D_IN), lambda i: (0, 0)),
            pl.BlockSpec((1, E + 1), lambda i: (0, 0)),
            pl.BlockSpec((E * R, D_IN), lambda i: (0, 0)),
            pl.BlockSpec((E * R, D_OUT), lambda i: (0, 0)),
        ],
        out_specs=pl.BlockSpec((TILE, D_OUT), lambda i: (i, 0)),
        out_shape=jax.ShapeDtypeStruct((t, D_OUT), jnp.float32),
        compiler_params=pltpu.CompilerParams(
            dimension_semantics=("parallel",),
        ),
    )(flat, rw_cat, rb_cat, lora_A2, b_flat)


def kernel(inputs, router_W, router_b, thr_W, thr_b, lora_A, lora_B):
    flat = inputs.reshape(-1, inputs.shape[-1])
    rw_cat = jnp.concatenate([router_W, thr_W], axis=0)
    rb_cat = jnp.concatenate([router_b, thr_b], axis=0).reshape(1, E + 1)
    b_flat = jnp.transpose(lora_B, (0, 2, 1)).reshape(E * R, D_OUT)
    out = _adamole(flat, rw_cat, rb_cat, lora_A.reshape(E * R, D_IN), b_flat)
    return out.reshape((*inputs.shape[:-1], D_OUT))
